# Initial kernel scaffold; baseline (speedup 1.0000x reference)
import jax
import jax.numpy as jnp
from jax.experimental import pallas as pl


def kernel(input_ids, embed_table):
    raise NotImplementedError("write your pallas kernel here")


# speedup vs baseline: 1.6699x; 1.6699x over previous
import functools

import jax
import jax.numpy as jnp
from jax import lax
from jax.experimental import pallas as pl
from jax.experimental.pallas import tpu as pltpu
from jax.experimental.pallas import tpu_sc as plsc

VOCAB = 32000
D_MODEL = 2048
NTOK = 4 * 4096

NC = 2
NS = 16
NW = NC * NS
BPW = NTOK // NW
CH = 16
NBUF = 2
NCHUNK = BPW // CH
assert NCHUNK % NBUF == 0 and NCHUNK // NBUF >= 2


@functools.cache
def _make_gather():
  mesh = plsc.VectorSubcoreMesh(core_axis_name="c", subcore_axis_name="s")

  @functools.partial(
      pl.kernel,
      mesh=mesh,
      out_type=jax.ShapeDtypeStruct((NTOK, D_MODEL), jnp.float32),
      scratch_types=[
          pltpu.VMEM((NCHUNK, CH), jnp.int32),
          pltpu.VMEM((NBUF, CH, D_MODEL), jnp.float32),
          pltpu.SemaphoreType.DMA((NBUF,)),
          pltpu.SemaphoreType.DMA((NBUF,)),
      ],
  )
  def emb_kernel(ids_hbm, table_hbm, out_hbm, idx_v, bufs, gsem, psem):
    wid = lax.axis_index("s") * NC + lax.axis_index("c")
    base = wid * BPW

    pltpu.sync_copy(ids_hbm.at[wid], idx_v)

    def start_gather(c, b):
      pltpu.async_copy(table_hbm.at[idx_v.at[c]], bufs.at[b], gsem.at[b])

    def start_put(c, b):
      pltpu.async_copy(
          bufs.at[b], out_hbm.at[pl.ds(base + c * CH, CH)], psem.at[b]
      )

    def wait_gather(b):
      pltpu.make_async_copy(table_hbm.at[idx_v.at[0]], bufs.at[b],
                            gsem.at[b]).wait()

    def wait_put(b):
      pltpu.make_async_copy(bufs.at[b], out_hbm.at[pl.ds(0, CH)],
                            psem.at[b]).wait()

    for b in range(NBUF):
      start_gather(b, b)

    def body(g, _):
      for b in range(NBUF):
        c = g + b
        wait_gather(b)
        start_put(c, b)
      for b in range(NBUF):
        wait_put(b)
        start_gather(g + NBUF + b, b)
      return ()

    lax.fori_loop(0, NCHUNK // NBUF - 1, lambda i, _: body(i * NBUF, _),
                  (), unroll=False)

    g_tail = NCHUNK - NBUF
    for b in range(NBUF):
      wait_gather(b)
      start_put(g_tail + b, b)
    for b in range(NBUF):
      wait_put(b)

  return emb_kernel


@jax.jit
def kernel(input_ids, embed_table):
  ids = input_ids.reshape(-1).astype(jnp.int32)
  ids_r = ids.reshape(NW, NCHUNK, CH)
  flat = _make_gather()(ids_r, embed_table)
  hidden = flat.reshape(input_ids.shape + (D_MODEL,))
  return (hidden, input_ids + 0)


# speedup vs baseline: 1.7456x; 1.0453x over previous
import functools

import jax
import jax.numpy as jnp
from jax import lax
from jax.experimental import pallas as pl
from jax.experimental.pallas import tpu as pltpu
from jax.experimental.pallas import tpu_sc as plsc

VOCAB = 32000
D_MODEL = 2048
NTOK = 4 * 4096

NC = 2
NS = 16
NW = NC * NS
BPW = NTOK // NW
CH = 8
NBUF = 4
LOOK = 2
NCHUNK = BPW // CH
assert (NCHUNK - 2 * LOOK) % NBUF == 0 and NCHUNK >= 2 * NBUF


def _make_gather():
  mesh = plsc.VectorSubcoreMesh(core_axis_name="c", subcore_axis_name="s")

  @functools.partial(
      pl.kernel,
      mesh=mesh,
      out_type=jax.ShapeDtypeStruct((NTOK, D_MODEL), jnp.float32),
      scratch_types=[
          pltpu.VMEM((NCHUNK, CH), jnp.int32),
          pltpu.VMEM((NBUF, CH, D_MODEL), jnp.float32),
          pltpu.SemaphoreType.DMA((NBUF,)),
          pltpu.SemaphoreType.DMA((NBUF,)),
      ],
  )
  def emb_kernel(ids_hbm, table_hbm, out_hbm, idx_v, bufs, gsem, psem):
    wid = lax.axis_index("s") * NC + lax.axis_index("c")
    base = wid * BPW

    pltpu.sync_copy(ids_hbm.at[wid], idx_v)

    def start_gather(c, b):
      pltpu.async_copy(table_hbm.at[idx_v.at[c]], bufs.at[b], gsem.at[b])

    def start_put(c, b):
      pltpu.async_copy(
          bufs.at[b], out_hbm.at[pl.ds(base + c * CH, CH)], psem.at[b]
      )

    def wait_gather(b):
      pltpu.make_async_copy(table_hbm.at[idx_v.at[0]], bufs.at[b],
                            gsem.at[b]).wait()

    def wait_put(b):
      pltpu.make_async_copy(bufs.at[b], out_hbm.at[pl.ds(0, CH)],
                            psem.at[b]).wait()

    for j in range(LOOK):
      start_gather(j, j)

    for c in range(LOOK):
      b = c % NBUF
      wait_gather(b)
      start_put(c, b)
      start_gather(c + LOOK, (c + LOOK) % NBUF)

    def step(c, b, bb):
      wait_gather(b)
      start_put(c, b)
      wait_put(bb)
      start_gather(c + LOOK, bb)

    def block(i, _):
      c0 = LOOK + i * NBUF
      for j in range(NBUF):
        b = (LOOK + j) % NBUF
        step(c0 + j, b, (b + LOOK) % NBUF)
      return ()

    lax.fori_loop(0, (NCHUNK - 2 * LOOK) // NBUF, block, (), unroll=False)

    for c in range(NCHUNK - LOOK, NCHUNK):
      b = c % NBUF
      wait_gather(b)
      start_put(c, b)

    for b in range(NBUF):
      wait_put(b)

  return emb_kernel


_GATHER_CACHE = {}


def _gather_fn():
  if "g" not in _GATHER_CACHE:
    _GATHER_CACHE["g"] = _make_gather()
  return _GATHER_CACHE["g"]


@jax.jit
def kernel(input_ids, embed_table):
  ids = input_ids.reshape(-1).astype(jnp.int32)
  ids_r = ids.reshape(NW, NCHUNK, CH)
  flat = _gather_fn()(ids_r, embed_table)
  hidden = flat.reshape(input_ids.shape + (D_MODEL,))
  return (hidden, input_ids + 0)


# speedup vs baseline: 1.7579x; 1.0070x over previous
import functools

import jax
import jax.numpy as jnp
from jax import lax
from jax.experimental import pallas as pl
from jax.experimental.pallas import tpu as pltpu
from jax.experimental.pallas import tpu_sc as plsc

VOCAB = 32000
D_MODEL = 2048
NTOK = 4 * 4096

NC = 2
NS = 16
NW = NC * NS
BPW = NTOK // NW
CH = 8
NBUF = 4
LOOK = 2
NCHUNK = BPW // CH
assert (NCHUNK - 2 * LOOK) % NBUF == 0 and NCHUNK >= 2 * NBUF


def _make_gather():
  mesh = plsc.VectorSubcoreMesh(core_axis_name="c", subcore_axis_name="s")

  @functools.partial(
      pl.kernel,
      mesh=mesh,
      out_type=jax.ShapeDtypeStruct((NTOK, D_MODEL), jnp.float32),
      scratch_types=[
          pltpu.VMEM((NCHUNK, CH), jnp.int32),
          pltpu.VMEM((NBUF, CH, D_MODEL), jnp.float32),
          pltpu.SemaphoreType.DMA((NBUF,)),
          pltpu.SemaphoreType.DMA((NBUF,)),
      ],
  )
  def emb_kernel(ids_hbm, table_hbm, out_hbm, idx_v, bufs, gsem, psem):
    wid = lax.axis_index("s") * NC + lax.axis_index("c")
    base = wid * BPW

    pltpu.sync_copy(ids_hbm.at[wid], idx_v)

    def start_gather(c, b):
      pltpu.async_copy(table_hbm.at[idx_v.at[c]], bufs.at[b], gsem.at[b])

    def start_put(c, b):
      pltpu.async_copy(
          bufs.at[b], out_hbm.at[pl.ds(base + c * CH, CH)], psem.at[b]
      )

    def wait_gather(b):
      pltpu.make_async_copy(table_hbm.at[idx_v.at[0]], bufs.at[b],
                            gsem.at[b]).wait()

    def wait_put(b):
      pltpu.make_async_copy(bufs.at[b], out_hbm.at[pl.ds(0, CH)],
                            psem.at[b]).wait()

    for j in range(LOOK):
      start_gather(j, j)

    for c in range(LOOK):
      b = c % NBUF
      start_gather(c + LOOK, (c + LOOK) % NBUF)
      wait_gather(b)
      start_put(c, b)

    def step(c, b, bb):
      wait_put(bb)
      start_gather(c + LOOK, bb)
      wait_gather(b)
      start_put(c, b)

    def block(i, _):
      c0 = LOOK + i * NBUF
      for j in range(NBUF):
        b = (LOOK + j) % NBUF
        step(c0 + j, b, (b + LOOK) % NBUF)
      return ()

    lax.fori_loop(0, (NCHUNK - 2 * LOOK) // NBUF, block, (), unroll=False)

    for c in range(NCHUNK - LOOK, NCHUNK):
      b = c % NBUF
      wait_gather(b)
      start_put(c, b)

    for b in range(NBUF):
      wait_put(b)

  return emb_kernel


_GATHER_CACHE = {}


def _gather_fn():
  if "g" not in _GATHER_CACHE:
    _GATHER_CACHE["g"] = _make_gather()
  return _GATHER_CACHE["g"]


@jax.jit
def kernel(input_ids, embed_table):
  ids = input_ids.reshape(-1).astype(jnp.int32)
  ids_r = ids.reshape(NW, NCHUNK, CH)
  flat = _gather_fn()(ids_r, embed_table)
  hidden = flat.reshape(input_ids.shape + (D_MODEL,))
  return (hidden, input_ids + 0)


# speedup vs baseline: 1.7652x; 1.0041x over previous
import functools

import jax
import jax.numpy as jnp
from jax import lax
from jax.experimental import pallas as pl
from jax.experimental.pallas import tpu as pltpu
from jax.experimental.pallas import tpu_sc as plsc

VOCAB = 32000
D_MODEL = 2048
NTOK = 4 * 4096

NC = 2
NS = 16
NW = NC * NS
BPW = NTOK // NW
CH = 8
NBUF = 4
LOOK = 2
NCHUNK = BPW // CH
assert (NCHUNK - 2 * LOOK) % NBUF == 0 and NCHUNK >= 2 * NBUF


def _make_gather():
  mesh = plsc.VectorSubcoreMesh(core_axis_name="c", subcore_axis_name="s")

  @functools.partial(
      pl.kernel,
      mesh=mesh,
      out_type=jax.ShapeDtypeStruct((NTOK, D_MODEL), jnp.float32),
      scratch_types=[
          pltpu.VMEM((NCHUNK, CH), jnp.int32),
          pltpu.VMEM((NBUF, CH, D_MODEL), jnp.float32),
          pltpu.SemaphoreType.DMA((NBUF,)),
      ],
  )
  def emb_kernel(ids_hbm, table_hbm, out_hbm, idx_v, bufs, gsem):
    wid = lax.axis_index("s") * NC + lax.axis_index("c")
    base = wid * BPW

    pltpu.sync_copy(ids_hbm.at[wid], idx_v)

    def start_gather(c, b):
      pltpu.async_copy(table_hbm.at[idx_v.at[c]], bufs.at[b], gsem.at[b])

    def wait_gather(b):
      pltpu.make_async_copy(table_hbm.at[idx_v.at[0]], bufs.at[b],
                            gsem.at[b]).wait()

    def sync_put(c, b):
      pltpu.sync_copy(bufs.at[b], out_hbm.at[pl.ds(base + c * CH, CH)])

    for j in range(NBUF - 1):
      start_gather(j, j)

    def step(c, b):
      wait_gather(b)
      start_gather(c + NBUF - 1, (b + NBUF - 1) % NBUF)
      sync_put(c, b)

    def block(i, _):
      c0 = i * NBUF
      for j in range(NBUF):
        step(c0 + j, j)
      return ()

    lax.fori_loop(0, NCHUNK // NBUF - 1, block, (), unroll=False)

    c0 = NCHUNK - NBUF
    start_gather(NCHUNK - 1, (NCHUNK - 1) % NBUF)
    for j in range(NBUF):
      c = c0 + j
      b = c % NBUF
      wait_gather(b)
      sync_put(c, b)

  return emb_kernel


_GATHER_CACHE = {}


def _gather_fn():
  if "g" not in _GATHER_CACHE:
    _GATHER_CACHE["g"] = _make_gather()
  return _GATHER_CACHE["g"]


@jax.jit
def kernel(input_ids, embed_table):
  ids = input_ids.reshape(-1).astype(jnp.int32)
  ids_r = ids.reshape(NW, NCHUNK, CH)
  flat = _gather_fn()(ids_r, embed_table)
  hidden = flat.reshape(input_ids.shape + (D_MODEL,))
  return (hidden, input_ids + 0)
